# Initial kernel scaffold; baseline (speedup 1.0000x reference)
import jax
import jax.numpy as jnp
from jax.experimental import pallas as pl


def kernel(x, edge_index, edge_attr, W_h, b_h, W_n, b_n, w_e, W_ft, b_ft):
    raise NotImplementedError("write your pallas kernel here")


# speedup vs baseline: 1.8467x; 1.8467x over previous
import functools

import jax
import jax.numpy as jnp
from jax import lax
from jax.experimental import pallas as pl
from jax.experimental.pallas import tpu as pltpu
from jax.experimental.pallas import tpu_sc as plsc

N = 10000
E = 320000
D = 128
ED = 16
H = 128

NC = 2
NS = 16
NW = NC * NS
EPW = E // NW
CH = 80
NSTEP = EPW // CH
KS = D // 16
HN = 5120
ACCR = 5248
RPT = ACCR // NS

_mesh = plsc.VectorSubcoreMesh(core_axis_name="c", subcore_axis_name="s")


def _ab_body(x_ref, wn1_ref, wn2_ref, bias_ref, a_ref, b_ref):
    x = x_ref[...]
    a_ref[...] = lax.dot_general(x, wn1_ref[...], (((1,), (1,)), ((), ())),
                                 preferred_element_type=jnp.float32) + bias_ref[...]
    b_ref[...] = lax.dot_general(x, wn2_ref[...], (((1,), (1,)), ((), ())),
                                 preferred_element_type=jnp.float32)


def _eh_body(ea_ref, wbig_ref, out_ref):
    out_ref[...] = jnp.dot(ea_ref[...], wbig_ref[...],
                           preferred_element_type=jnp.float32)


def _softmax_body(s_ref, o_ref):
    s = s_ref[...]
    m = jnp.max(s)
    ex = jnp.exp(s - m)
    o_ref[...] = ex / jnp.sum(ex)


def _final_body(x_ref, p_ref, wft_ref, bft_ref, o_ref):
    out0 = x_ref[...] + p_ref[0] + p_ref[1]
    o_ref[...] = out0 + lax.dot_general(
        out0, wft_ref[...], (((1,), (1,)), ((), ())),
        preferred_element_type=jnp.float32) + bft_ref[...]


def _scores_body(a_hbm, b_hbm, eh_hbm, src_hbm, dst_hbm, we_hbm, scores_hbm,
                 idx_s, idx_d, a_buf, b_buf, eh_buf, we_v, scores_v, tmp):
    c = lax.axis_index("c")
    s = lax.axis_index("s")
    wid = s * NC + c
    base = wid * EPW

    pltpu.sync_copy(src_hbm.at[wid], idx_s)
    pltpu.sync_copy(dst_hbm.at[wid], idx_d)
    pltpu.sync_copy(we_hbm, we_v)
    we_regs = [we_v[pl.ds(16 * k, 16)] for k in range(KS)]

    def step_body(step, carry):
        pltpu.sync_copy(a_hbm.at[idx_s.at[step]], a_buf)
        pltpu.sync_copy(b_hbm.at[idx_d.at[step]], b_buf)
        pltpu.sync_copy(eh_hbm.at[pl.ds(base + step * CH, CH)], eh_buf)

        lane_iota = lax.iota(jnp.int32, 16)

        def grp_body(g, carry2):
            for l in range(16):
                e = g * 16 + l
                acc = jnp.zeros((16,), jnp.float32)
                for k in range(KS):
                    sl = pl.ds(16 * k, 16)
                    v = eh_buf[e, sl] + a_buf[e, sl] + b_buf[e, sl]
                    acc = acc + jnp.maximum(v, 0.0) * we_regs[k]
                tmp[l, :] = acc
            svec = jnp.zeros((16,), jnp.float32)
            for k in range(16):
                col_idx = jnp.full((16,), k, jnp.int32)
                svec = svec + plsc.load_gather(tmp, [lane_iota, col_idx])
            scores_v[pl.ds(step * CH + g * 16, 16)] = svec
            return carry2

        lax.fori_loop(0, CH // 16, grp_body, 0)
        return carry

    lax.fori_loop(0, NSTEP, step_body, 0)
    pltpu.sync_copy(scores_v, scores_hbm.at[pl.ds(base, EPW)])


def _scatter_body(x_hbm, src_hbm, dst_hbm, alpha_hbm, zeros_hbm,
                  part_hbm, idx_s, idx_d, alpha_v, rows, idx_adj, acc):
    c = lax.axis_index("c")
    s = lax.axis_index("s")
    wid = s * NC + c
    base = wid * EPW
    row0 = s * RPT

    pltpu.sync_copy(src_hbm.at[wid], idx_s)
    pltpu.sync_copy(dst_hbm.at[wid], idx_d)
    pltpu.sync_copy(alpha_hbm.at[pl.ds(base, EPW)], alpha_v)

    for h in range(2):
        pltpu.sync_copy(zeros_hbm.at[pl.ds(row0, RPT)], acc.at[pl.ds(row0, RPT)])
        plsc.subcore_barrier()

        def step_body(step, carry):
            pltpu.sync_copy(x_hbm.at[idx_d.at[step]], rows)

            def grp_body(g, carry2):
                iv = idx_s[step, pl.ds(g * 16, 16)] - (h * HN)
                valid = (iv >= 0) & (iv < HN)
                idx_adj[pl.ds(g * 16, 16)] = jnp.where(valid, iv, HN)
                return carry2

            lax.fori_loop(0, CH // 16, grp_body, 0)

            def edge_body(e, carry2):
                idxv = jnp.full((16,), step * CH + e, jnp.int32)
                neg_a = jnp.float32(0.0) - plsc.load_gather(alpha_v, [idxv])
                for k in range(KS):
                    sl = pl.ds(16 * k, 16)
                    rows[e, sl] = rows[e, sl] * neg_a
                return carry2

            lax.fori_loop(0, CH, edge_body, 0)
            pltpu.sync_copy(rows, acc.at[idx_adj], add=True)
            return carry

        lax.fori_loop(0, NSTEP, step_body, 0)

        plsc.subcore_barrier()
        pltpu.sync_copy(acc.at[pl.ds(row0, RPT)],
                        part_hbm.at[c, h, pl.ds(row0, RPT)])
        plsc.subcore_barrier()


def kernel(x, edge_index, edge_attr, W_h, b_h, W_n, b_n, w_e, W_ft, b_ft):
    src = edge_index[0]
    dst = edge_index[1]
    src3d = src.reshape(NW, NSTEP, CH)
    dst3d = dst.reshape(NW, NSTEP, CH)
    wn1 = W_n[:, :D]
    wn2 = W_n[:, D:]
    bias = b_h + b_n
    wev = w_e[:, 0]

    wbig = jnp.kron(jnp.eye(8, dtype=jnp.float32), W_h.T)
    ea_rs = edge_attr.reshape(E // 8, 8 * ED)

    a_mat, b_mat = pl.pallas_call(
        _ab_body,
        out_shape=(jax.ShapeDtypeStruct((N, H), jnp.float32),
                   jax.ShapeDtypeStruct((N, H), jnp.float32)),
    )(x, wn1, wn2, bias)

    BE = 2000
    eh = pl.pallas_call(
        _eh_body,
        grid=(E // 8 // BE,),
        in_specs=[pl.BlockSpec((BE, 8 * ED), lambda i: (i, 0)),
                  pl.BlockSpec((8 * ED, 8 * H), lambda i: (0, 0))],
        out_specs=pl.BlockSpec((BE, 8 * H), lambda i: (i, 0)),
        out_shape=jax.ShapeDtypeStruct((E // 8, 8 * H), jnp.float32),
    )(ea_rs, wbig)
    eh = eh.reshape(E, H)

    scores = pl.kernel(
        _scores_body,
        out_type=jax.ShapeDtypeStruct((E,), jnp.float32),
        mesh=_mesh,
        compiler_params=pltpu.CompilerParams(needs_layout_passes=False),
        scratch_types=[
            pltpu.VMEM((NSTEP, CH), jnp.int32),
            pltpu.VMEM((NSTEP, CH), jnp.int32),
            pltpu.VMEM((CH, H), jnp.float32),
            pltpu.VMEM((CH, H), jnp.float32),
            pltpu.VMEM((CH, H), jnp.float32),
            pltpu.VMEM((H,), jnp.float32),
            pltpu.VMEM((EPW,), jnp.float32),
            pltpu.VMEM((16, 16), jnp.float32),
        ],
    )(a_mat, b_mat, eh, src3d, dst3d, wev)

    alpha2d = pl.pallas_call(
        _softmax_body,
        out_shape=jax.ShapeDtypeStruct((E // H, H), jnp.float32),
    )(scores.reshape(E // H, H))
    alpha = alpha2d.reshape(E)

    zeros = jnp.zeros((ACCR, D), jnp.float32)
    parts = pl.kernel(
        _scatter_body,
        out_type=jax.ShapeDtypeStruct((NC, 2, ACCR, D), jnp.float32),
        mesh=_mesh,
        compiler_params=pltpu.CompilerParams(needs_layout_passes=False),
        scratch_types=[
            pltpu.VMEM((NSTEP, CH), jnp.int32),
            pltpu.VMEM((NSTEP, CH), jnp.int32),
            pltpu.VMEM((EPW,), jnp.float32),
            pltpu.VMEM((CH, D), jnp.float32),
            pltpu.VMEM((CH,), jnp.int32),
            pltpu.VMEM_SHARED((ACCR, D), jnp.float32),
        ],
    )(x, src3d, dst3d, alpha, zeros)

    parts_full = jnp.concatenate([parts[:, 0, :HN, :], parts[:, 1, :N - HN, :]],
                                 axis=1)

    out = pl.pallas_call(
        _final_body,
        out_shape=jax.ShapeDtypeStruct((N, D), jnp.float32),
    )(x, parts_full, W_ft, b_ft)

    return (out, alpha)


# speedup vs baseline: 2.9997x; 1.6244x over previous
import functools

import jax
import jax.numpy as jnp
from jax import lax
from jax.experimental import pallas as pl
from jax.experimental.pallas import tpu as pltpu
from jax.experimental.pallas import tpu_sc as plsc

N = 10000
E = 320000
D = 128
ED = 16
H = 128

NC = 2
NS = 16
NW = NC * NS
EPW = E // NW
CH = 80
NSTEP = EPW // CH
KS = D // 16
HN = 5120
ACCR = 5248
RPT = ACCR // NS

_mesh = plsc.VectorSubcoreMesh(core_axis_name="c", subcore_axis_name="s")


def _ab_body(x_ref, wn1_ref, wn2_ref, bias_ref, a_ref, b_ref):
    x = x_ref[...]
    a_ref[...] = lax.dot_general(x, wn1_ref[...], (((1,), (1,)), ((), ())),
                                 preferred_element_type=jnp.float32) + bias_ref[...]
    b_ref[...] = lax.dot_general(x, wn2_ref[...], (((1,), (1,)), ((), ())),
                                 preferred_element_type=jnp.float32)


def _eh_body(ea_ref, wbig_ref, out_ref):
    out_ref[...] = jnp.dot(ea_ref[...], wbig_ref[...],
                           preferred_element_type=jnp.float32)


def _softmax_body(s_ref, o_ref):
    s = s_ref[...]
    m = jnp.max(s)
    ex = jnp.exp(s - m)
    o_ref[...] = ex / jnp.sum(ex)


def _final_body(x_ref, p_ref, wft_ref, bft_ref, o_ref):
    out0 = x_ref[...] + p_ref[0] + p_ref[1]
    o_ref[...] = out0 + lax.dot_general(
        out0, wft_ref[...], (((1,), (1,)), ((), ())),
        preferred_element_type=jnp.float32) + bft_ref[...]


def _scores_body(a_hbm, b_hbm, eh_hbm, src_hbm, dst_hbm, we_hbm, scores_hbm,
                 idx_s, idx_d, a_buf, b_buf, eh_buf, we_v, scores_v, tmp,
                 sem0, sem1):
    c = lax.axis_index("c")
    s = lax.axis_index("s")
    wid = s * NC + c
    base = wid * EPW

    pltpu.sync_copy(src_hbm.at[wid], idx_s)
    pltpu.sync_copy(dst_hbm.at[wid], idx_d)
    pltpu.sync_copy(we_hbm, we_v)
    we_regs = [we_v[pl.ds(16 * k, 16)] for k in range(KS)]
    lane_iota = lax.iota(jnp.int32, 16)
    slots = ((a_buf.at[0], b_buf.at[0], eh_buf.at[0], sem0),
             (a_buf.at[1], b_buf.at[1], eh_buf.at[1], sem1))

    def start(step, slot):
        av, bv, ev, sem = slots[slot]
        pltpu.async_copy(a_hbm.at[idx_s.at[step]], av, sem)
        pltpu.async_copy(b_hbm.at[idx_d.at[step]], bv, sem)
        pltpu.async_copy(eh_hbm.at[pl.ds(base + step * CH, CH)], ev, sem)

    def wait(slot):
        av, bv, ev, sem = slots[slot]
        dummy = eh_hbm.at[pl.ds(0, CH)]
        pltpu.make_async_copy(dummy, av, sem).wait()
        pltpu.make_async_copy(dummy, bv, sem).wait()
        pltpu.make_async_copy(dummy, ev, sem).wait()

    def compute(step, slot):
        av, bv, ev, _ = slots[slot]

        def grp_body(g, carry2):
            for l in range(16):
                e = g * 16 + l
                acc = jnp.zeros((16,), jnp.float32)
                for k in range(KS):
                    sl = pl.ds(16 * k, 16)
                    v = ev[e, sl] + av[e, sl] + bv[e, sl]
                    acc = acc + jnp.maximum(v, 0.0) * we_regs[k]
                tmp[l, :] = acc
            svec = jnp.zeros((16,), jnp.float32)
            for k in range(16):
                col_idx = jnp.full((16,), k, jnp.int32)
                svec = svec + plsc.load_gather(tmp, [lane_iota, col_idx])
            scores_v[pl.ds(step * CH + g * 16, 16)] = svec
            return carry2

        lax.fori_loop(0, CH // 16, grp_body, 0)

    start(0, 0)

    def pair_body(i, carry):
        s0 = 2 * i
        wait(0)
        start(s0 + 1, 1)
        compute(s0, 0)
        wait(1)
        start(s0 + 2, 0)
        compute(s0 + 1, 1)
        return carry

    lax.fori_loop(0, (NSTEP - 1) // 2, pair_body, 0)
    wait(0)
    compute(NSTEP - 1, 0)
    pltpu.sync_copy(scores_v, scores_hbm.at[pl.ds(base, EPW)])


def _scatter_body(x_hbm, src_hbm, dst_hbm, alpha_hbm, zeros_hbm,
                  part_hbm, idx_s, idx_d, alpha_v, rows, idx_adj, acc,
                  sem0, sem1):
    c = lax.axis_index("c")
    s = lax.axis_index("s")
    wid = s * NC + c
    base = wid * EPW
    row0 = s * RPT

    pltpu.sync_copy(src_hbm.at[wid], idx_s)
    pltpu.sync_copy(dst_hbm.at[wid], idx_d)
    pltpu.sync_copy(alpha_hbm.at[pl.ds(base, EPW)], alpha_v)

    slots = ((rows.at[0], sem0), (rows.at[1], sem1))

    def start(step, slot):
        rv, sem = slots[slot]
        pltpu.async_copy(x_hbm.at[idx_d.at[step]], rv, sem)

    def wait(slot):
        rv, sem = slots[slot]
        pltpu.make_async_copy(x_hbm.at[pl.ds(0, CH)], rv, sem).wait()

    for h in range(2):
        pltpu.sync_copy(zeros_hbm.at[pl.ds(row0, RPT)], acc.at[pl.ds(row0, RPT)])
        plsc.subcore_barrier()

        def process(step, slot):
            rv, _ = slots[slot]

            def grp_body(g, carry2):
                iv = idx_s[step, pl.ds(g * 16, 16)] - (h * HN)
                valid = (iv >= 0) & (iv < HN)
                idx_adj[pl.ds(g * 16, 16)] = jnp.where(valid, iv, HN)
                return carry2

            lax.fori_loop(0, CH // 16, grp_body, 0)

            def edge_body(e, carry2):
                idxv = jnp.full((16,), step * CH + e, jnp.int32)
                neg_a = jnp.float32(0.0) - plsc.load_gather(alpha_v, [idxv])
                for k in range(KS):
                    sl = pl.ds(16 * k, 16)
                    rv[e, sl] = rv[e, sl] * neg_a
                return carry2

            lax.fori_loop(0, CH, edge_body, 0)
            pltpu.sync_copy(rv, acc.at[idx_adj], add=True)

        start(0, 0)

        def pair_body(i, carry):
            s0 = 2 * i
            wait(0)
            start(s0 + 1, 1)
            process(s0, 0)
            wait(1)
            start(s0 + 2, 0)
            process(s0 + 1, 1)
            return carry

        lax.fori_loop(0, (NSTEP - 1) // 2, pair_body, 0)
        wait(0)
        process(NSTEP - 1, 0)

        plsc.subcore_barrier()
        pltpu.sync_copy(acc.at[pl.ds(row0, RPT)],
                        part_hbm.at[c, h, pl.ds(row0, RPT)])
        plsc.subcore_barrier()


def kernel(x, edge_index, edge_attr, W_h, b_h, W_n, b_n, w_e, W_ft, b_ft):
    src = edge_index[0]
    dst = edge_index[1]
    src3d = src.reshape(NW, NSTEP, CH)
    dst3d = dst.reshape(NW, NSTEP, CH)
    wn1 = W_n[:, :D]
    wn2 = W_n[:, D:]
    bias = b_h + b_n
    wev = w_e[:, 0]

    wbig = jnp.kron(jnp.eye(8, dtype=jnp.float32), W_h.T)
    ea_rs = edge_attr.reshape(E // 8, 8 * ED)

    a_mat, b_mat = pl.pallas_call(
        _ab_body,
        out_shape=(jax.ShapeDtypeStruct((N, H), jnp.float32),
                   jax.ShapeDtypeStruct((N, H), jnp.float32)),
    )(x, wn1, wn2, bias)

    BE = 2000
    eh = pl.pallas_call(
        _eh_body,
        grid=(E // 8 // BE,),
        in_specs=[pl.BlockSpec((BE, 8 * ED), lambda i: (i, 0)),
                  pl.BlockSpec((8 * ED, 8 * H), lambda i: (0, 0))],
        out_specs=pl.BlockSpec((BE, 8 * H), lambda i: (i, 0)),
        out_shape=jax.ShapeDtypeStruct((E // 8, 8 * H), jnp.float32),
    )(ea_rs, wbig)
    eh = eh.reshape(E, H)

    scores = pl.kernel(
        _scores_body,
        out_type=jax.ShapeDtypeStruct((E,), jnp.float32),
        mesh=_mesh,
        compiler_params=pltpu.CompilerParams(needs_layout_passes=False),
        scratch_types=[
            pltpu.VMEM((NSTEP, CH), jnp.int32),
            pltpu.VMEM((NSTEP, CH), jnp.int32),
            pltpu.VMEM((2, CH, H), jnp.float32),
            pltpu.VMEM((2, CH, H), jnp.float32),
            pltpu.VMEM((2, CH, H), jnp.float32),
            pltpu.VMEM((H,), jnp.float32),
            pltpu.VMEM((EPW,), jnp.float32),
            pltpu.VMEM((16, 16), jnp.float32),
            pltpu.SemaphoreType.DMA,
            pltpu.SemaphoreType.DMA,
        ],
    )(a_mat, b_mat, eh, src3d, dst3d, wev)

    alpha2d = pl.pallas_call(
        _softmax_body,
        out_shape=jax.ShapeDtypeStruct((E // H, H), jnp.float32),
    )(scores.reshape(E // H, H))
    alpha = alpha2d.reshape(E)

    zeros = jnp.zeros((ACCR, D), jnp.float32)
    parts = pl.kernel(
        _scatter_body,
        out_type=jax.ShapeDtypeStruct((NC, 2, ACCR, D), jnp.float32),
        mesh=_mesh,
        compiler_params=pltpu.CompilerParams(needs_layout_passes=False),
        scratch_types=[
            pltpu.VMEM((NSTEP, CH), jnp.int32),
            pltpu.VMEM((NSTEP, CH), jnp.int32),
            pltpu.VMEM((EPW,), jnp.float32),
            pltpu.VMEM((2, CH, D), jnp.float32),
            pltpu.VMEM((CH,), jnp.int32),
            pltpu.VMEM_SHARED((ACCR, D), jnp.float32),
            pltpu.SemaphoreType.DMA,
            pltpu.SemaphoreType.DMA,
        ],
    )(x, src3d, dst3d, alpha, zeros)

    parts_full = jnp.concatenate([parts[:, 0, :HN, :], parts[:, 1, :N - HN, :]],
                                 axis=1)

    out = pl.pallas_call(
        _final_body,
        out_shape=jax.ShapeDtypeStruct((N, D), jnp.float32),
    )(x, parts_full, W_ft, b_ft)

    return (out, alpha)


# speedup vs baseline: 3.6819x; 1.2274x over previous
import functools

import jax
import jax.numpy as jnp
from jax import lax
from jax.experimental import pallas as pl
from jax.experimental.pallas import tpu as pltpu
from jax.experimental.pallas import tpu_sc as plsc

N = 10000
E = 320000
D = 128
ED = 16
H = 128

NC = 2
NS = 16
NW = NC * NS
EPW = E // NW
CH = 80
NSTEP = EPW // CH
KS = D // 16
RPT = 624

_mesh = plsc.VectorSubcoreMesh(core_axis_name="c", subcore_axis_name="s")


def _ab_body(x_ref, wn1_ref, wn2_ref, bias_ref, a_ref, b_ref):
    x = x_ref[...]
    a_ref[...] = lax.dot_general(x, wn1_ref[...], (((1,), (1,)), ((), ())),
                                 preferred_element_type=jnp.float32) + bias_ref[...]
    b_ref[...] = lax.dot_general(x, wn2_ref[...], (((1,), (1,)), ((), ())),
                                 preferred_element_type=jnp.float32)


def _eh_body(ea_ref, wbig_ref, out_ref):
    out_ref[...] = jnp.dot(ea_ref[...], wbig_ref[...],
                           preferred_element_type=jnp.float32)


def _softmax_body(s_ref, o_ref):
    s = s_ref[...]
    m = jnp.max(s)
    ex = jnp.exp(s - m)
    o_ref[...] = ex / jnp.sum(ex)


def _final_body(x_ref, p_ref, wft_ref, bft_ref, o_ref):
    out0 = x_ref[...] + p_ref[0] + p_ref[1]
    o_ref[...] = out0 + lax.dot_general(
        out0, wft_ref[...], (((1,), (1,)), ((), ())),
        preferred_element_type=jnp.float32) + bft_ref[...]


def _scores_body(a_hbm, b_hbm, eh_hbm, src_hbm, dst_hbm, we_hbm, scores_hbm,
                 idx_s, idx_d, a_buf, b_buf, eh_buf, we_v, scores_v, tmp,
                 sem0, sem1):
    c = lax.axis_index("c")
    s = lax.axis_index("s")
    wid = s * NC + c
    base = wid * EPW

    pltpu.sync_copy(src_hbm.at[wid], idx_s)
    pltpu.sync_copy(dst_hbm.at[wid], idx_d)
    pltpu.sync_copy(we_hbm, we_v)
    we_regs = [we_v[pl.ds(16 * k, 16)] for k in range(KS)]
    lane_iota = lax.iota(jnp.int32, 16)
    slots = ((a_buf.at[0], b_buf.at[0], eh_buf.at[0], sem0),
             (a_buf.at[1], b_buf.at[1], eh_buf.at[1], sem1))

    def start(step, slot):
        av, bv, ev, sem = slots[slot]
        pltpu.async_copy(a_hbm.at[idx_s.at[step]], av, sem)
        pltpu.async_copy(b_hbm.at[idx_d.at[step]], bv, sem)
        pltpu.async_copy(eh_hbm.at[pl.ds(base + step * CH, CH)], ev, sem)

    def wait(slot):
        av, bv, ev, sem = slots[slot]
        dummy = eh_hbm.at[pl.ds(0, CH)]
        pltpu.make_async_copy(dummy, av, sem).wait()
        pltpu.make_async_copy(dummy, bv, sem).wait()
        pltpu.make_async_copy(dummy, ev, sem).wait()

    def compute(step, slot):
        av, bv, ev, _ = slots[slot]

        def grp_body(g, carry2):
            for l in range(16):
                e = g * 16 + l
                acc = jnp.zeros((16,), jnp.float32)
                for k in range(KS):
                    sl = pl.ds(16 * k, 16)
                    v = ev[e, sl] + av[e, sl] + bv[e, sl]
                    acc = acc + jnp.maximum(v, 0.0) * we_regs[k]
                tmp[l, :] = acc
            svec = jnp.zeros((16,), jnp.float32)
            for k in range(16):
                col_idx = jnp.full((16,), k, jnp.int32)
                svec = svec + plsc.load_gather(tmp, [lane_iota, col_idx])
            scores_v[pl.ds(step * CH + g * 16, 16)] = svec
            return carry2

        lax.fori_loop(0, CH // 16, grp_body, 0)

    start(0, 0)

    def pair_body(i, carry):
        s0 = 2 * i
        wait(0)
        start(s0 + 1, 1)
        compute(s0, 0)
        wait(1)
        start(s0 + 2, 0)
        compute(s0 + 1, 1)
        return carry

    lax.fori_loop(0, (NSTEP - 1) // 2, pair_body, 0)
    wait(0)
    compute(NSTEP - 1, 0)
    pltpu.sync_copy(scores_v, scores_hbm.at[pl.ds(base, EPW)])


def _scatter_body(x_hbm, src_hbm, dst_hbm, alpha_hbm, zeros_hbm,
                  part_hbm, idx_sb, idx_d, alpha_b, rows, acc, sem0, sem1):
    c = lax.axis_index("c")
    s = lax.axis_index("s")
    wid = s * NC + c
    base = wid * EPW
    row0 = s * RPT

    pltpu.sync_copy(dst_hbm.at[wid], idx_d)

    pltpu.sync_copy(zeros_hbm.at[pl.ds(row0, RPT)], acc.at[pl.ds(row0, RPT)])

    @pl.when(s == 0)
    def _():
        pltpu.sync_copy(zeros_hbm.at[pl.ds(NS * RPT, N - NS * RPT)],
                        acc.at[pl.ds(NS * RPT, N - NS * RPT)])

    slots = ((rows.at[0], alpha_b.at[0], idx_sb.at[0], sem0),
             (rows.at[1], alpha_b.at[1], idx_sb.at[1], sem1))

    def start(step, slot):
        rv, av, iv, sem = slots[slot]
        pltpu.async_copy(x_hbm.at[idx_d.at[step]], rv, sem)
        pltpu.async_copy(alpha_hbm.at[pl.ds(base + step * CH, CH)], av, sem)
        pltpu.async_copy(src_hbm.at[wid, step], iv, sem)

    def wait(slot):
        rv, av, iv, sem = slots[slot]
        pltpu.make_async_copy(x_hbm.at[pl.ds(0, CH)], rv, sem).wait()
        pltpu.make_async_copy(alpha_hbm.at[pl.ds(0, CH)], av, sem).wait()
        pltpu.make_async_copy(src_hbm.at[0, 0], iv, sem).wait()

    plsc.subcore_barrier()

    def process(step, slot):
        rv, av, iv, _ = slots[slot]

        def edge_body(e, carry2):
            idxv = jnp.full((16,), e, jnp.int32)
            neg_a = jnp.float32(0.0) - plsc.load_gather(av, [idxv])
            for k in range(KS):
                sl = pl.ds(16 * k, 16)
                rv[e, sl] = rv[e, sl] * neg_a
            return carry2

        lax.fori_loop(0, CH, edge_body, 0)
        pltpu.sync_copy(rv, acc.at[iv], add=True)

    start(0, 0)

    def pair_body(i, carry):
        s0 = 2 * i
        wait(0)
        start(s0 + 1, 1)
        process(s0, 0)
        wait(1)
        start(s0 + 2, 0)
        process(s0 + 1, 1)
        return carry

    lax.fori_loop(0, (NSTEP - 1) // 2, pair_body, 0)
    wait(0)
    process(NSTEP - 1, 0)

    plsc.subcore_barrier()
    pltpu.sync_copy(acc.at[pl.ds(row0, RPT)], part_hbm.at[c, pl.ds(row0, RPT)])

    @pl.when(s == 0)
    def _():
        pltpu.sync_copy(acc.at[pl.ds(NS * RPT, N - NS * RPT)],
                        part_hbm.at[c, pl.ds(NS * RPT, N - NS * RPT)])


def kernel(x, edge_index, edge_attr, W_h, b_h, W_n, b_n, w_e, W_ft, b_ft):
    src = edge_index[0]
    dst = edge_index[1]
    src3d = src.reshape(NW, NSTEP, CH)
    dst3d = dst.reshape(NW, NSTEP, CH)
    wn1 = W_n[:, :D]
    wn2 = W_n[:, D:]
    bias = b_h + b_n
    wev = w_e[:, 0]

    wbig = jnp.kron(jnp.eye(8, dtype=jnp.float32), W_h.T)
    ea_rs = edge_attr.reshape(E // 8, 8 * ED)

    a_mat, b_mat = pl.pallas_call(
        _ab_body,
        out_shape=(jax.ShapeDtypeStruct((N, H), jnp.float32),
                   jax.ShapeDtypeStruct((N, H), jnp.float32)),
    )(x, wn1, wn2, bias)

    BE = 2000
    eh = pl.pallas_call(
        _eh_body,
        grid=(E // 8 // BE,),
        in_specs=[pl.BlockSpec((BE, 8 * ED), lambda i: (i, 0)),
                  pl.BlockSpec((8 * ED, 8 * H), lambda i: (0, 0))],
        out_specs=pl.BlockSpec((BE, 8 * H), lambda i: (i, 0)),
        out_shape=jax.ShapeDtypeStruct((E // 8, 8 * H), jnp.float32),
    )(ea_rs, wbig)
    eh = eh.reshape(E, H)

    scores = pl.kernel(
        _scores_body,
        out_type=jax.ShapeDtypeStruct((E,), jnp.float32),
        mesh=_mesh,
        compiler_params=pltpu.CompilerParams(needs_layout_passes=False),
        scratch_types=[
            pltpu.VMEM((NSTEP, CH), jnp.int32),
            pltpu.VMEM((NSTEP, CH), jnp.int32),
            pltpu.VMEM((2, CH, H), jnp.float32),
            pltpu.VMEM((2, CH, H), jnp.float32),
            pltpu.VMEM((2, CH, H), jnp.float32),
            pltpu.VMEM((H,), jnp.float32),
            pltpu.VMEM((EPW,), jnp.float32),
            pltpu.VMEM((16, 16), jnp.float32),
            pltpu.SemaphoreType.DMA,
            pltpu.SemaphoreType.DMA,
        ],
    )(a_mat, b_mat, eh, src3d, dst3d, wev)

    alpha2d = pl.pallas_call(
        _softmax_body,
        out_shape=jax.ShapeDtypeStruct((E // H, H), jnp.float32),
    )(scores.reshape(E // H, H))
    alpha = alpha2d.reshape(E)

    zeros = jnp.zeros((N, D), jnp.float32)
    parts = pl.kernel(
        _scatter_body,
        out_type=jax.ShapeDtypeStruct((NC, N, D), jnp.float32),
        mesh=_mesh,
        compiler_params=pltpu.CompilerParams(needs_layout_passes=False),
        scratch_types=[
            pltpu.VMEM((2, CH), jnp.int32),
            pltpu.VMEM((NSTEP, CH), jnp.int32),
            pltpu.VMEM((2, CH), jnp.float32),
            pltpu.VMEM((2, CH, D), jnp.float32),
            pltpu.VMEM_SHARED((N, D), jnp.float32),
            pltpu.SemaphoreType.DMA,
            pltpu.SemaphoreType.DMA,
        ],
    )(x, src3d, dst3d, alpha, zeros)

    out = pl.pallas_call(
        _final_body,
        out_shape=jax.ShapeDtypeStruct((N, D), jnp.float32),
    )(x, parts, W_ft, b_ft)

    return (out, alpha)


# speedup vs baseline: 3.7151x; 1.0090x over previous
import functools

import jax
import jax.numpy as jnp
from jax import lax
from jax.experimental import pallas as pl
from jax.experimental.pallas import tpu as pltpu
from jax.experimental.pallas import tpu_sc as plsc

N = 10000
E = 320000
D = 128
ED = 16
H = 128

NC = 2
NS = 16
NW = NC * NS
EPW = E // NW
CH = 80
NSTEP = EPW // CH
KS = D // 16
RPT = 624

_mesh = plsc.VectorSubcoreMesh(core_axis_name="c", subcore_axis_name="s")


def _eh_body(ea_ref, wbig_ref, x_ref, wn1_ref, wn2_ref, bias_ref,
             out_ref, a_ref, b_ref):
    out_ref[...] = jnp.dot(ea_ref[...], wbig_ref[...],
                           preferred_element_type=jnp.float32)

    @pl.when(pl.program_id(0) == 0)
    def _():
        x = x_ref[...]
        a_ref[...] = lax.dot_general(
            x, wn1_ref[...], (((1,), (1,)), ((), ())),
            preferred_element_type=jnp.float32) + bias_ref[...]
        b_ref[...] = lax.dot_general(
            x, wn2_ref[...], (((1,), (1,)), ((), ())),
            preferred_element_type=jnp.float32)


def _softmax_body(s_ref, o_ref):
    s = s_ref[...]
    m = jnp.max(s)
    ex = jnp.exp(s - m)
    o_ref[...] = ex / jnp.sum(ex)


def _final_body(x_ref, p_ref, wft_ref, bft_ref, o_ref):
    out0 = x_ref[...] + p_ref[0] + p_ref[1]
    o_ref[...] = out0 + lax.dot_general(
        out0, wft_ref[...], (((1,), (1,)), ((), ())),
        preferred_element_type=jnp.float32) + bft_ref[...]


def _scores_body(a_hbm, b_hbm, eh_hbm, src_hbm, dst_hbm, we_hbm, scores_hbm,
                 idx_s, idx_d, a_buf, b_buf, eh_buf, we_v, scores_v, tmp,
                 sem0, sem1):
    c = lax.axis_index("c")
    s = lax.axis_index("s")
    wid = s * NC + c
    base = wid * EPW

    pltpu.sync_copy(src_hbm.at[wid], idx_s)
    pltpu.sync_copy(dst_hbm.at[wid], idx_d)
    pltpu.sync_copy(we_hbm, we_v)
    we_regs = [we_v[pl.ds(16 * k, 16)] for k in range(KS)]
    lane_iota = lax.iota(jnp.int32, 16)
    slots = ((a_buf.at[0], b_buf.at[0], eh_buf.at[0], sem0),
             (a_buf.at[1], b_buf.at[1], eh_buf.at[1], sem1))

    def start(step, slot):
        av, bv, ev, sem = slots[slot]
        pltpu.async_copy(a_hbm.at[idx_s.at[step]], av, sem)
        pltpu.async_copy(b_hbm.at[idx_d.at[step]], bv, sem)
        pltpu.async_copy(eh_hbm.at[pl.ds(base + step * CH, CH)], ev, sem)

    def wait(slot):
        av, bv, ev, sem = slots[slot]
        dummy = eh_hbm.at[pl.ds(0, CH)]
        pltpu.make_async_copy(dummy, av, sem).wait()
        pltpu.make_async_copy(dummy, bv, sem).wait()
        pltpu.make_async_copy(dummy, ev, sem).wait()

    def compute(step, slot):
        av, bv, ev, _ = slots[slot]

        def grp_body(g, carry2):
            for l in range(16):
                e = g * 16 + l
                acc = jnp.zeros((16,), jnp.float32)
                for k in range(KS):
                    sl = pl.ds(16 * k, 16)
                    v = ev[e, sl] + av[e, sl] + bv[e, sl]
                    acc = acc + jnp.maximum(v, 0.0) * we_regs[k]
                tmp[l, :] = acc
            svec = jnp.zeros((16,), jnp.float32)
            for k in range(16):
                col_idx = jnp.full((16,), k, jnp.int32)
                svec = svec + plsc.load_gather(tmp, [lane_iota, col_idx])
            scores_v[pl.ds(step * CH + g * 16, 16)] = svec
            return carry2

        lax.fori_loop(0, CH // 16, grp_body, 0)

    start(0, 0)

    def pair_body(i, carry):
        s0 = 2 * i
        wait(0)
        start(s0 + 1, 1)
        compute(s0, 0)
        wait(1)
        start(s0 + 2, 0)
        compute(s0 + 1, 1)
        return carry

    lax.fori_loop(0, (NSTEP - 1) // 2, pair_body, 0)
    wait(0)
    compute(NSTEP - 1, 0)
    pltpu.sync_copy(scores_v, scores_hbm.at[pl.ds(base, EPW)])


def _scatter_body(x_hbm, src_hbm, dst_hbm, alpha_hbm,
                  part_hbm, idx_sb, idx_d, alpha_b, rows, acc, sem0, sem1):
    c = lax.axis_index("c")
    s = lax.axis_index("s")
    wid = s * NC + c
    base = wid * EPW
    row0 = s * RPT

    pltpu.sync_copy(dst_hbm.at[wid], idx_d)

    zv = jnp.zeros((16,), jnp.float32)

    def zb_body(r, carry):
        for k in range(KS):
            rows[0, r, pl.ds(16 * k, 16)] = zv
        return carry

    lax.fori_loop(0, CH, zb_body, 0)
    zsrc = rows.at[0]
    for j in range(RPT // CH):
        pltpu.sync_copy(zsrc, acc.at[pl.ds(row0 + j * CH, CH)])
    rem = RPT - (RPT // CH) * CH
    if rem:
        pltpu.sync_copy(zsrc.at[pl.ds(0, rem)],
                        acc.at[pl.ds(row0 + RPT - rem, rem)])

    @pl.when(s == 0)
    def _():
        pltpu.sync_copy(zsrc.at[pl.ds(0, N - NS * RPT)],
                        acc.at[pl.ds(NS * RPT, N - NS * RPT)])

    slots = ((rows.at[0], alpha_b.at[0], idx_sb.at[0], sem0),
             (rows.at[1], alpha_b.at[1], idx_sb.at[1], sem1))

    def start(step, slot):
        rv, av, iv, sem = slots[slot]
        pltpu.async_copy(x_hbm.at[idx_d.at[step]], rv, sem)
        pltpu.async_copy(alpha_hbm.at[pl.ds(base + step * CH, CH)], av, sem)
        pltpu.async_copy(src_hbm.at[wid, step], iv, sem)

    def wait(slot):
        rv, av, iv, sem = slots[slot]
        pltpu.make_async_copy(x_hbm.at[pl.ds(0, CH)], rv, sem).wait()
        pltpu.make_async_copy(alpha_hbm.at[pl.ds(0, CH)], av, sem).wait()
        pltpu.make_async_copy(src_hbm.at[0, 0], iv, sem).wait()

    plsc.subcore_barrier()

    def process(step, slot):
        rv, av, iv, _ = slots[slot]

        def edge_body(e, carry2):
            idxv = jnp.full((16,), e, jnp.int32)
            neg_a = jnp.float32(0.0) - plsc.load_gather(av, [idxv])
            for k in range(KS):
                sl = pl.ds(16 * k, 16)
                rv[e, sl] = rv[e, sl] * neg_a
            return carry2

        lax.fori_loop(0, CH, edge_body, 0)
        pltpu.sync_copy(rv, acc.at[iv], add=True)

    start(0, 0)

    def pair_body(i, carry):
        s0 = 2 * i
        wait(0)
        start(s0 + 1, 1)
        process(s0, 0)
        wait(1)
        start(s0 + 2, 0)
        process(s0 + 1, 1)
        return carry

    lax.fori_loop(0, (NSTEP - 1) // 2, pair_body, 0)
    wait(0)
    process(NSTEP - 1, 0)

    plsc.subcore_barrier()
    pltpu.sync_copy(acc.at[pl.ds(row0, RPT)], part_hbm.at[c, pl.ds(row0, RPT)])

    @pl.when(s == 0)
    def _():
        pltpu.sync_copy(acc.at[pl.ds(NS * RPT, N - NS * RPT)],
                        part_hbm.at[c, pl.ds(NS * RPT, N - NS * RPT)])


def kernel(x, edge_index, edge_attr, W_h, b_h, W_n, b_n, w_e, W_ft, b_ft):
    src = edge_index[0]
    dst = edge_index[1]
    src3d = src.reshape(NW, NSTEP, CH)
    dst3d = dst.reshape(NW, NSTEP, CH)
    wn1 = W_n[:, :D]
    wn2 = W_n[:, D:]
    bias = b_h + b_n
    wev = w_e[:, 0]

    wbig = jnp.kron(jnp.eye(8, dtype=jnp.float32), W_h.T)
    ea_rs = edge_attr.reshape(E // 8, 8 * ED)

    BE = 2000
    eh, a_mat, b_mat = pl.pallas_call(
        _eh_body,
        grid=(E // 8 // BE,),
        in_specs=[pl.BlockSpec((BE, 8 * ED), lambda i: (i, 0)),
                  pl.BlockSpec((8 * ED, 8 * H), lambda i: (0, 0)),
                  pl.BlockSpec((N, D), lambda i: (0, 0)),
                  pl.BlockSpec((H, D), lambda i: (0, 0)),
                  pl.BlockSpec((H, D), lambda i: (0, 0)),
                  pl.BlockSpec((H,), lambda i: (0,))],
        out_specs=(pl.BlockSpec((BE, 8 * H), lambda i: (i, 0)),
                   pl.BlockSpec((N, H), lambda i: (0, 0)),
                   pl.BlockSpec((N, H), lambda i: (0, 0))),
        out_shape=(jax.ShapeDtypeStruct((E // 8, 8 * H), jnp.float32),
                   jax.ShapeDtypeStruct((N, H), jnp.float32),
                   jax.ShapeDtypeStruct((N, H), jnp.float32)),
    )(ea_rs, wbig, x, wn1, wn2, bias)
    eh = eh.reshape(E, H)

    scores = pl.kernel(
        _scores_body,
        out_type=jax.ShapeDtypeStruct((E,), jnp.float32),
        mesh=_mesh,
        compiler_params=pltpu.CompilerParams(needs_layout_passes=False),
        scratch_types=[
            pltpu.VMEM((NSTEP, CH), jnp.int32),
            pltpu.VMEM((NSTEP, CH), jnp.int32),
            pltpu.VMEM((2, CH, H), jnp.float32),
            pltpu.VMEM((2, CH, H), jnp.float32),
            pltpu.VMEM((2, CH, H), jnp.float32),
            pltpu.VMEM((H,), jnp.float32),
            pltpu.VMEM((EPW,), jnp.float32),
            pltpu.VMEM((16, 16), jnp.float32),
            pltpu.SemaphoreType.DMA,
            pltpu.SemaphoreType.DMA,
        ],
    )(a_mat, b_mat, eh, src3d, dst3d, wev)

    alpha2d = pl.pallas_call(
        _softmax_body,
        out_shape=jax.ShapeDtypeStruct((E // H, H), jnp.float32),
    )(scores.reshape(E // H, H))
    alpha = alpha2d.reshape(E)

    parts = pl.kernel(
        _scatter_body,
        out_type=jax.ShapeDtypeStruct((NC, N, D), jnp.float32),
        mesh=_mesh,
        compiler_params=pltpu.CompilerParams(needs_layout_passes=False),
        scratch_types=[
            pltpu.VMEM((2, CH), jnp.int32),
            pltpu.VMEM((NSTEP, CH), jnp.int32),
            pltpu.VMEM((2, CH), jnp.float32),
            pltpu.VMEM((2, CH, D), jnp.float32),
            pltpu.VMEM_SHARED((N, D), jnp.float32),
            pltpu.SemaphoreType.DMA,
            pltpu.SemaphoreType.DMA,
        ],
    )(x, src3d, dst3d, alpha)

    out = pl.pallas_call(
        _final_body,
        out_shape=jax.ShapeDtypeStruct((N, D), jnp.float32),
    )(x, parts, W_ft, b_ft)

    return (out, alpha)


# speedup vs baseline: 4.4552x; 1.1992x over previous
import functools

import jax
import jax.numpy as jnp
from jax import lax
from jax.experimental import pallas as pl
from jax.experimental.pallas import tpu as pltpu
from jax.experimental.pallas import tpu_sc as plsc

N = 10000
E = 320000
D = 128
ED = 16
H = 128

NC = 2
NS = 16
NW = NC * NS
EPW = E // NW
CH = 80
NSTEP = EPW // CH
KS = D // 16
RPT = 624

_mesh = plsc.VectorSubcoreMesh(core_axis_name="c", subcore_axis_name="s")


def _eh_body(ea_ref, wpad_ref, x_ref, wn1_ref, wn2_ref, bias_ref,
             out_ref, a_ref, b_ref):
    ea = jnp.pad(ea_ref[...], ((0, 0), (0, D - ED)))
    out_ref[...] = jnp.dot(ea, wpad_ref[...],
                           preferred_element_type=jnp.float32)

    @pl.when(pl.program_id(0) == 0)
    def _():
        x = x_ref[...]
        a_ref[...] = lax.dot_general(
            x, wn1_ref[...], (((1,), (1,)), ((), ())),
            preferred_element_type=jnp.float32) + bias_ref[...]
        b_ref[...] = lax.dot_general(
            x, wn2_ref[...], (((1,), (1,)), ((), ())),
            preferred_element_type=jnp.float32)


def _softmax_body(s_ref, o_ref):
    s = s_ref[...]
    m = jnp.max(s)
    ex = jnp.exp(s - m)
    o_ref[...] = ex / jnp.sum(ex)


def _final_body(x_ref, p_ref, wft_ref, bft_ref, o_ref):
    out0 = x_ref[...] + p_ref[0] + p_ref[1]
    o_ref[...] = out0 + lax.dot_general(
        out0, wft_ref[...], (((1,), (1,)), ((), ())),
        preferred_element_type=jnp.float32) + bft_ref[...]


def _scores_body(a_hbm, b_hbm, eh_hbm, src_hbm, dst_hbm, we_hbm, scores_hbm,
                 idx_s, idx_d, a_buf, b_buf, eh_buf, we_v, scores_v, tmp,
                 sem0, sem1):
    c = lax.axis_index("c")
    s = lax.axis_index("s")
    wid = s * NC + c
    base = wid * EPW

    pltpu.sync_copy(src_hbm.at[wid], idx_s)
    pltpu.sync_copy(dst_hbm.at[wid], idx_d)
    pltpu.sync_copy(we_hbm, we_v)
    we_regs = [we_v[pl.ds(16 * k, 16)] for k in range(KS)]
    lane_iota = lax.iota(jnp.int32, 16)
    slots = ((a_buf.at[0], b_buf.at[0], eh_buf.at[0], sem0),
             (a_buf.at[1], b_buf.at[1], eh_buf.at[1], sem1))

    def start(step, slot):
        av, bv, ev, sem = slots[slot]
        pltpu.async_copy(a_hbm.at[idx_s.at[step]], av, sem)
        pltpu.async_copy(b_hbm.at[idx_d.at[step]], bv, sem)
        pltpu.async_copy(eh_hbm.at[pl.ds(base + step * CH, CH)], ev, sem)

    def wait(slot):
        av, bv, ev, sem = slots[slot]
        dummy = eh_hbm.at[pl.ds(0, CH)]
        pltpu.make_async_copy(dummy, av, sem).wait()
        pltpu.make_async_copy(dummy, bv, sem).wait()
        pltpu.make_async_copy(dummy, ev, sem).wait()

    def compute(step, slot):
        av, bv, ev, _ = slots[slot]

        def grp_body(g, carry2):
            for l in range(16):
                e = g * 16 + l
                acc = jnp.zeros((16,), jnp.float32)
                for k in range(KS):
                    sl = pl.ds(16 * k, 16)
                    v = ev[e, sl] + av[e, sl] + bv[e, sl]
                    acc = acc + jnp.maximum(v, 0.0) * we_regs[k]
                tmp[l, :] = acc
            svec = jnp.zeros((16,), jnp.float32)
            for k in range(16):
                col_idx = jnp.full((16,), k, jnp.int32)
                svec = svec + plsc.load_gather(tmp, [lane_iota, col_idx])
            scores_v[pl.ds(step * CH + g * 16, 16)] = svec
            return carry2

        lax.fori_loop(0, CH // 16, grp_body, 0)

    start(0, 0)

    def pair_body(i, carry):
        s0 = 2 * i
        wait(0)
        start(s0 + 1, 1)
        compute(s0, 0)
        wait(1)
        start(s0 + 2, 0)
        compute(s0 + 1, 1)
        return carry

    lax.fori_loop(0, (NSTEP - 1) // 2, pair_body, 0)
    wait(0)
    compute(NSTEP - 1, 0)
    pltpu.sync_copy(scores_v, scores_hbm.at[pl.ds(base, EPW)])


def _scatter_body(x_hbm, src_hbm, dst_hbm, alpha_hbm,
                  part_hbm, idx_sb, idx_d, alpha_b, rows, acc, sem0, sem1):
    c = lax.axis_index("c")
    s = lax.axis_index("s")
    wid = s * NC + c
    base = wid * EPW
    row0 = s * RPT

    pltpu.sync_copy(dst_hbm.at[wid], idx_d)

    zv = jnp.zeros((16,), jnp.float32)

    def zb_body(r, carry):
        for k in range(KS):
            rows[0, r, pl.ds(16 * k, 16)] = zv
        return carry

    lax.fori_loop(0, CH, zb_body, 0)
    zsrc = rows.at[0]
    for j in range(RPT // CH):
        pltpu.sync_copy(zsrc, acc.at[pl.ds(row0 + j * CH, CH)])
    rem = RPT - (RPT // CH) * CH
    if rem:
        pltpu.sync_copy(zsrc.at[pl.ds(0, rem)],
                        acc.at[pl.ds(row0 + RPT - rem, rem)])

    @pl.when(s == 0)
    def _():
        pltpu.sync_copy(zsrc.at[pl.ds(0, N - NS * RPT)],
                        acc.at[pl.ds(NS * RPT, N - NS * RPT)])

    slots = ((rows.at[0], alpha_b.at[0], idx_sb.at[0], sem0),
             (rows.at[1], alpha_b.at[1], idx_sb.at[1], sem1))

    def start(step, slot):
        rv, av, iv, sem = slots[slot]
        pltpu.async_copy(x_hbm.at[idx_d.at[step]], rv, sem)
        pltpu.async_copy(alpha_hbm.at[pl.ds(base + step * CH, CH)], av, sem)
        pltpu.async_copy(src_hbm.at[wid, step], iv, sem)

    def wait(slot):
        rv, av, iv, sem = slots[slot]
        pltpu.make_async_copy(x_hbm.at[pl.ds(0, CH)], rv, sem).wait()
        pltpu.make_async_copy(alpha_hbm.at[pl.ds(0, CH)], av, sem).wait()
        pltpu.make_async_copy(src_hbm.at[0, 0], iv, sem).wait()

    plsc.subcore_barrier()

    def process(step, slot):
        rv, av, iv, _ = slots[slot]

        def edge_body(e, carry2):
            idxv = jnp.full((16,), e, jnp.int32)
            neg_a = jnp.float32(0.0) - plsc.load_gather(av, [idxv])
            for k in range(KS):
                sl = pl.ds(16 * k, 16)
                rv[e, sl] = rv[e, sl] * neg_a
            return carry2

        lax.fori_loop(0, CH, edge_body, 0)
        pltpu.sync_copy(rv, acc.at[iv], add=True)

    start(0, 0)

    def pair_body(i, carry):
        s0 = 2 * i
        wait(0)
        start(s0 + 1, 1)
        process(s0, 0)
        wait(1)
        start(s0 + 2, 0)
        process(s0 + 1, 1)
        return carry

    lax.fori_loop(0, (NSTEP - 1) // 2, pair_body, 0)
    wait(0)
    process(NSTEP - 1, 0)

    plsc.subcore_barrier()
    pltpu.sync_copy(acc.at[pl.ds(row0, RPT)], part_hbm.at[c, pl.ds(row0, RPT)])

    @pl.when(s == 0)
    def _():
        pltpu.sync_copy(acc.at[pl.ds(NS * RPT, N - NS * RPT)],
                        part_hbm.at[c, pl.ds(NS * RPT, N - NS * RPT)])


def kernel(x, edge_index, edge_attr, W_h, b_h, W_n, b_n, w_e, W_ft, b_ft):
    src = edge_index[0]
    dst = edge_index[1]
    src3d = src.reshape(NW, NSTEP, CH)
    dst3d = dst.reshape(NW, NSTEP, CH)
    wn1 = W_n[:, :D]
    wn2 = W_n[:, D:]
    bias = b_h + b_n
    wev = w_e[:, 0]

    wpad = jnp.pad(W_h.T, ((0, D - ED), (0, 0)))

    BE = 8000
    eh, a_mat, b_mat = pl.pallas_call(
        _eh_body,
        grid=(E // BE,),
        in_specs=[pl.BlockSpec((BE, ED), lambda i: (i, 0)),
                  pl.BlockSpec((D, H), lambda i: (0, 0)),
                  pl.BlockSpec((N, D), lambda i: (0, 0)),
                  pl.BlockSpec((H, D), lambda i: (0, 0)),
                  pl.BlockSpec((H, D), lambda i: (0, 0)),
                  pl.BlockSpec((H,), lambda i: (0,))],
        out_specs=(pl.BlockSpec((BE, H), lambda i: (i, 0)),
                   pl.BlockSpec((N, H), lambda i: (0, 0)),
                   pl.BlockSpec((N, H), lambda i: (0, 0))),
        out_shape=(jax.ShapeDtypeStruct((E, H), jnp.float32),
                   jax.ShapeDtypeStruct((N, H), jnp.float32),
                   jax.ShapeDtypeStruct((N, H), jnp.float32)),
    )(edge_attr, wpad, x, wn1, wn2, bias)

    scores = pl.kernel(
        _scores_body,
        out_type=jax.ShapeDtypeStruct((E,), jnp.float32),
        mesh=_mesh,
        compiler_params=pltpu.CompilerParams(needs_layout_passes=False),
        scratch_types=[
            pltpu.VMEM((NSTEP, CH), jnp.int32),
            pltpu.VMEM((NSTEP, CH), jnp.int32),
            pltpu.VMEM((2, CH, H), jnp.float32),
            pltpu.VMEM((2, CH, H), jnp.float32),
            pltpu.VMEM((2, CH, H), jnp.float32),
            pltpu.VMEM((H,), jnp.float32),
            pltpu.VMEM((EPW,), jnp.float32),
            pltpu.VMEM((16, 16), jnp.float32),
            pltpu.SemaphoreType.DMA,
            pltpu.SemaphoreType.DMA,
        ],
    )(a_mat, b_mat, eh, src3d, dst3d, wev)

    alpha2d = pl.pallas_call(
        _softmax_body,
        out_shape=jax.ShapeDtypeStruct((E // H, H), jnp.float32),
    )(scores.reshape(E // H, H))
    alpha = alpha2d.reshape(E)

    parts = pl.kernel(
        _scatter_body,
        out_type=jax.ShapeDtypeStruct((NC, N, D), jnp.float32),
        mesh=_mesh,
        compiler_params=pltpu.CompilerParams(needs_layout_passes=False),
        scratch_types=[
            pltpu.VMEM((2, CH), jnp.int32),
            pltpu.VMEM((NSTEP, CH), jnp.int32),
            pltpu.VMEM((2, CH), jnp.float32),
            pltpu.VMEM((2, CH, D), jnp.float32),
            pltpu.VMEM_SHARED((N, D), jnp.float32),
            pltpu.SemaphoreType.DMA,
            pltpu.SemaphoreType.DMA,
        ],
    )(x, src3d, dst3d, alpha)

    out = pl.pallas_call(
        _final_body,
        out_shape=jax.ShapeDtypeStruct((N, D), jnp.float32),
    )(x, parts, W_ft, b_ft)

    return (out, alpha)


# speedup vs baseline: 4.4635x; 1.0019x over previous
import functools

import jax
import jax.numpy as jnp
from jax import lax
from jax.experimental import pallas as pl
from jax.experimental.pallas import tpu as pltpu
from jax.experimental.pallas import tpu_sc as plsc

N = 10000
E = 320000
D = 128
ED = 16
H = 128

NC = 2
NS = 16
NW = NC * NS
EPW = E // NW
CH = 80
NSTEP = EPW // CH
E2A = 163840
E2B = E - E2A
KS = D // 16
RPT = 624

_mesh = plsc.VectorSubcoreMesh(core_axis_name="c", subcore_axis_name="s")


def _eh_body(ea_ref, wpad_ref, x_ref, wn1_ref, wn2_ref, bias_ref,
             out_ref, a_ref, b_ref):
    ea = jnp.pad(ea_ref[...], ((0, 0), (0, D - ED)))
    out_ref[...] = jnp.dot(ea, wpad_ref[...],
                           preferred_element_type=jnp.float32)

    @pl.when(pl.program_id(0) == 0)
    def _():
        x = x_ref[...]
        a_ref[...] = lax.dot_general(
            x, wn1_ref[...], (((1,), (1,)), ((), ())),
            preferred_element_type=jnp.float32) + bias_ref[...]
        b_ref[...] = lax.dot_general(
            x, wn2_ref[...], (((1,), (1,)), ((), ())),
            preferred_element_type=jnp.float32)


def _eh2_body(ea_ref, wpad_ref, out_ref):
    ea = jnp.pad(ea_ref[...], ((0, 0), (0, D - ED)))
    out_ref[...] = jnp.dot(ea, wpad_ref[...],
                           preferred_element_type=jnp.float32)


def _softmax_body(s1_ref, s2_ref, o_ref):
    s1 = s1_ref[...]
    s2 = s2_ref[...]
    m = jnp.maximum(jnp.max(s1), jnp.max(s2))
    e1 = jnp.exp(s1 - m)
    e2 = jnp.exp(s2 - m)
    inv = 1.0 / (jnp.sum(e1) + jnp.sum(e2))
    o_ref[pl.ds(0, E2A // H), :] = e1 * inv
    o_ref[pl.ds(E2A // H, E2B // H), :] = e2 * inv


def _final_body(x_ref, p_ref, wft_ref, bft_ref, o_ref):
    out0 = x_ref[...] + p_ref[0] + p_ref[1]
    o_ref[...] = out0 + lax.dot_general(
        out0, wft_ref[...], (((1,), (1,)), ((), ())),
        preferred_element_type=jnp.float32) + bft_ref[...]


def _make_scores_body(epw, nstep):
    def _scores_body(a_hbm, b_hbm, eh_hbm, src_hbm, dst_hbm, we_hbm,
                     scores_hbm, idx_s, idx_d, a_buf, b_buf, eh_buf, we_v,
                     scores_v, tmp, sem0, sem1):
        c = lax.axis_index("c")
        s = lax.axis_index("s")
        wid = s * NC + c
        base = wid * epw

        pltpu.sync_copy(src_hbm.at[wid], idx_s)
        pltpu.sync_copy(dst_hbm.at[wid], idx_d)
        pltpu.sync_copy(we_hbm, we_v)
        we_regs = [we_v[pl.ds(16 * k, 16)] for k in range(KS)]
        lane_iota = lax.iota(jnp.int32, 16)
        slots = ((a_buf.at[0], b_buf.at[0], eh_buf.at[0], sem0),
                 (a_buf.at[1], b_buf.at[1], eh_buf.at[1], sem1))

        def start(step, slot):
            av, bv, ev, sem = slots[slot]
            pltpu.async_copy(a_hbm.at[idx_s.at[step]], av, sem)
            pltpu.async_copy(b_hbm.at[idx_d.at[step]], bv, sem)
            pltpu.async_copy(eh_hbm.at[pl.ds(base + step * CH, CH)], ev, sem)

        def wait(slot):
            av, bv, ev, sem = slots[slot]
            dummy = eh_hbm.at[pl.ds(0, CH)]
            pltpu.make_async_copy(dummy, av, sem).wait()
            pltpu.make_async_copy(dummy, bv, sem).wait()
            pltpu.make_async_copy(dummy, ev, sem).wait()

        def compute(step, slot):
            av, bv, ev, _ = slots[slot]

            def grp_body(g, carry2):
                for l in range(16):
                    e = g * 16 + l
                    acc = jnp.zeros((16,), jnp.float32)
                    for k in range(KS):
                        sl = pl.ds(16 * k, 16)
                        v = ev[e, sl] + av[e, sl] + bv[e, sl]
                        acc = acc + jnp.maximum(v, 0.0) * we_regs[k]
                    tmp[l, :] = acc
                svec = jnp.zeros((16,), jnp.float32)
                for k in range(16):
                    col_idx = jnp.full((16,), k, jnp.int32)
                    svec = svec + plsc.load_gather(tmp, [lane_iota, col_idx])
                scores_v[pl.ds(step * CH + g * 16, 16)] = svec
                return carry2

            lax.fori_loop(0, CH // 16, grp_body, 0)

        start(0, 0)

        def pair_body(i, carry):
            s0 = 2 * i
            wait(0)
            start(s0 + 1, 1)
            compute(s0, 0)
            wait(1)
            start(s0 + 2, 0)
            compute(s0 + 1, 1)
            return carry

        lax.fori_loop(0, (nstep - 1) // 2, pair_body, 0)
        if nstep % 2 == 1:
            wait(0)
            compute(nstep - 1, 0)
        else:
            wait(0)
            start(nstep - 1, 1)
            compute(nstep - 2, 0)
            wait(1)
            compute(nstep - 1, 1)
        pltpu.sync_copy(scores_v, scores_hbm.at[pl.ds(base, epw)])

    return _scores_body


def _scatter_body(x_hbm, src_hbm, dst_hbm, alpha_hbm,
                  part_hbm, idx_sb, idx_d, alpha_b, rows, acc, sem0, sem1):
    c = lax.axis_index("c")
    s = lax.axis_index("s")
    wid = s * NC + c
    base = wid * EPW
    row0 = s * RPT

    pltpu.sync_copy(dst_hbm.at[wid], idx_d)

    zv = jnp.zeros((16,), jnp.float32)

    def zb_body(r, carry):
        for k in range(KS):
            rows[0, r, pl.ds(16 * k, 16)] = zv
        return carry

    lax.fori_loop(0, CH, zb_body, 0)
    zsrc = rows.at[0]
    for j in range(RPT // CH):
        pltpu.sync_copy(zsrc, acc.at[pl.ds(row0 + j * CH, CH)])
    rem = RPT - (RPT // CH) * CH
    if rem:
        pltpu.sync_copy(zsrc.at[pl.ds(0, rem)],
                        acc.at[pl.ds(row0 + RPT - rem, rem)])

    @pl.when(s == 0)
    def _():
        pltpu.sync_copy(zsrc.at[pl.ds(0, N - NS * RPT)],
                        acc.at[pl.ds(NS * RPT, N - NS * RPT)])

    slots = ((rows.at[0], alpha_b.at[0], idx_sb.at[0], sem0),
             (rows.at[1], alpha_b.at[1], idx_sb.at[1], sem1))

    def start(step, slot):
        rv, av, iv, sem = slots[slot]
        pltpu.async_copy(x_hbm.at[idx_d.at[step]], rv, sem)
        pltpu.async_copy(alpha_hbm.at[pl.ds(base + step * CH, CH)], av, sem)
        pltpu.async_copy(src_hbm.at[wid, step], iv, sem)

    def wait(slot):
        rv, av, iv, sem = slots[slot]
        pltpu.make_async_copy(x_hbm.at[pl.ds(0, CH)], rv, sem).wait()
        pltpu.make_async_copy(alpha_hbm.at[pl.ds(0, CH)], av, sem).wait()
        pltpu.make_async_copy(src_hbm.at[0, 0], iv, sem).wait()

    plsc.subcore_barrier()

    def process(step, slot):
        rv, av, iv, _ = slots[slot]

        def edge_body(e, carry2):
            idxv = jnp.full((16,), e, jnp.int32)
            neg_a = jnp.float32(0.0) - plsc.load_gather(av, [idxv])
            for k in range(KS):
                sl = pl.ds(16 * k, 16)
                rv[e, sl] = rv[e, sl] * neg_a
            return carry2

        lax.fori_loop(0, CH, edge_body, 0)
        pltpu.sync_copy(rv, acc.at[iv], add=True)

    start(0, 0)

    def pair_body(i, carry):
        s0 = 2 * i
        wait(0)
        start(s0 + 1, 1)
        process(s0, 0)
        wait(1)
        start(s0 + 2, 0)
        process(s0 + 1, 1)
        return carry

    lax.fori_loop(0, (NSTEP - 1) // 2, pair_body, 0)
    wait(0)
    process(NSTEP - 1, 0)

    plsc.subcore_barrier()
    pltpu.sync_copy(acc.at[pl.ds(row0, RPT)], part_hbm.at[c, pl.ds(row0, RPT)])

    @pl.when(s == 0)
    def _():
        pltpu.sync_copy(acc.at[pl.ds(NS * RPT, N - NS * RPT)],
                        part_hbm.at[c, pl.ds(NS * RPT, N - NS * RPT)])


def kernel(x, edge_index, edge_attr, W_h, b_h, W_n, b_n, w_e, W_ft, b_ft):
    src = edge_index[0]
    dst = edge_index[1]
    src3d = src.reshape(NW, NSTEP, CH)
    dst3d = dst.reshape(NW, NSTEP, CH)
    wn1 = W_n[:, :D]
    wn2 = W_n[:, D:]
    bias = b_h + b_n
    wev = w_e[:, 0]

    wpad = jnp.pad(W_h.T, ((0, D - ED), (0, 0)))

    BE = 2560
    eh1, a_mat, b_mat = pl.pallas_call(
        _eh_body,
        grid=(E2A // BE,),
        in_specs=[pl.BlockSpec((BE, ED), lambda i: (i, 0)),
                  pl.BlockSpec((D, H), lambda i: (0, 0)),
                  pl.BlockSpec((N, D), lambda i: (0, 0)),
                  pl.BlockSpec((H, D), lambda i: (0, 0)),
                  pl.BlockSpec((H, D), lambda i: (0, 0)),
                  pl.BlockSpec((H,), lambda i: (0,))],
        out_specs=(pl.BlockSpec((BE, H), lambda i: (i, 0)),
                   pl.BlockSpec((N, H), lambda i: (0, 0)),
                   pl.BlockSpec((N, H), lambda i: (0, 0))),
        out_shape=(jax.ShapeDtypeStruct((E2A, H), jnp.float32),
                   jax.ShapeDtypeStruct((N, H), jnp.float32),
                   jax.ShapeDtypeStruct((N, H), jnp.float32)),
    )(edge_attr, wpad, x, wn1, wn2, bias)

    off = E2A // BE
    eh2 = pl.pallas_call(
        _eh2_body,
        grid=(E2B // BE,),
        in_specs=[pl.BlockSpec((BE, ED), lambda i: (i + off, 0)),
                  pl.BlockSpec((D, H), lambda i: (0, 0))],
        out_specs=pl.BlockSpec((BE, H), lambda i: (i, 0)),
        out_shape=jax.ShapeDtypeStruct((E2B, H), jnp.float32),
    )(edge_attr, wpad)

    def scores_call(epw, nstep, eh_h, src_h, dst_h):
        return pl.kernel(
            _make_scores_body(epw, nstep),
            out_type=jax.ShapeDtypeStruct((epw * NW,), jnp.float32),
            mesh=_mesh,
            compiler_params=pltpu.CompilerParams(needs_layout_passes=False),
            scratch_types=[
                pltpu.VMEM((nstep, CH), jnp.int32),
                pltpu.VMEM((nstep, CH), jnp.int32),
                pltpu.VMEM((2, CH, H), jnp.float32),
                pltpu.VMEM((2, CH, H), jnp.float32),
                pltpu.VMEM((2, CH, H), jnp.float32),
                pltpu.VMEM((H,), jnp.float32),
                pltpu.VMEM((epw,), jnp.float32),
                pltpu.VMEM((16, 16), jnp.float32),
                pltpu.SemaphoreType.DMA,
                pltpu.SemaphoreType.DMA,
            ],
        )(a_mat, b_mat, eh_h, src_h, dst_h, wev)

    srcA = src[:E2A].reshape(NW, E2A // NW // CH, CH)
    dstA = dst[:E2A].reshape(NW, E2A // NW // CH, CH)
    srcB = src[E2A:].reshape(NW, E2B // NW // CH, CH)
    dstB = dst[E2A:].reshape(NW, E2B // NW // CH, CH)
    scores1 = scores_call(E2A // NW, E2A // NW // CH, eh1, srcA, dstA)
    scores2 = scores_call(E2B // NW, E2B // NW // CH, eh2, srcB, dstB)

    alpha2d = pl.pallas_call(
        _softmax_body,
        out_shape=jax.ShapeDtypeStruct((E // H, H), jnp.float32),
    )(scores1.reshape(E2A // H, H), scores2.reshape(E2B // H, H))
    alpha = alpha2d.reshape(E)

    parts = pl.kernel(
        _scatter_body,
        out_type=jax.ShapeDtypeStruct((NC, N, D), jnp.float32),
        mesh=_mesh,
        compiler_params=pltpu.CompilerParams(needs_layout_passes=False),
        scratch_types=[
            pltpu.VMEM((2, CH), jnp.int32),
            pltpu.VMEM((NSTEP, CH), jnp.int32),
            pltpu.VMEM((2, CH), jnp.float32),
            pltpu.VMEM((2, CH, D), jnp.float32),
            pltpu.VMEM_SHARED((N, D), jnp.float32),
            pltpu.SemaphoreType.DMA,
            pltpu.SemaphoreType.DMA,
        ],
    )(x, src3d, dst3d, alpha)

    out = pl.pallas_call(
        _final_body,
        out_shape=jax.ShapeDtypeStruct((N, D), jnp.float32),
    )(x, parts, W_ft, b_ft)

    return (out, alpha)
